# Initial kernel scaffold; baseline (speedup 1.0000x reference)
import jax
import jax.numpy as jnp
from jax.experimental import pallas as pl


def kernel(x, Wr, Wg, Wu, Wd):
    raise NotImplementedError("write your pallas kernel here")


# speedup vs baseline: 1.5580x; 1.5580x over previous
import functools

import jax
import jax.numpy as jnp
from jax.experimental import pallas as pl
from jax.experimental.pallas import tpu as pltpu

EMBED = 768
FFN_D = 3072
NE = 8
NTOK = 2048
FT = 512
NFT = FFN_D // FT


def _router_body(x_ref, wrt_ref, comb_ref, idx_ref, w_ref, loss_ref):
    x = x_ref[...]
    wrt = wrt_ref[...]
    logits = jnp.dot(x, wrt, preferred_element_type=jnp.float32)
    ids = jax.lax.broadcasted_iota(jnp.int32, logits.shape, 1)
    m1 = jnp.max(logits, axis=1, keepdims=True)
    i1 = jnp.min(jnp.where(logits == m1, ids, NE), axis=1, keepdims=True)
    l2 = jnp.where(ids == i1, -jnp.inf, logits)
    m2 = jnp.max(l2, axis=1, keepdims=True)
    i2 = jnp.min(jnp.where(l2 == m2, ids, NE), axis=1, keepdims=True)
    t = jnp.exp(m2 - m1)
    w1 = 1.0 / (1.0 + t)
    w2 = 1.0 - w1
    comb_ref[...] = jnp.where(ids == i1, w1, 0.0) + jnp.where(ids == i2, w2, 0.0)
    idx_ref[...] = jnp.concatenate([i1, i2], axis=1)
    w_ref[...] = jnp.concatenate([w1, w2], axis=1)
    ex = jnp.exp(logits - m1)
    probs = ex / jnp.sum(ex, axis=1, keepdims=True)
    usage = jnp.sum(probs, axis=0, keepdims=True) * (1.0 / NTOK)
    loss_ref[...] = NE * jnp.sum(usage * usage, axis=1, keepdims=True)


def _router(x_flat, Wr):
    wrt = Wr.T
    return pl.pallas_call(
        _router_body,
        out_shape=(
            jax.ShapeDtypeStruct((NTOK, NE), jnp.float32),
            jax.ShapeDtypeStruct((NTOK, 2), jnp.int32),
            jax.ShapeDtypeStruct((NTOK, 2), jnp.float32),
            jax.ShapeDtypeStruct((1, 1), jnp.float32),
        ),
    )(x_flat, wrt)


def _dense_ffn_body(x_ref, comb_ref, wg_ref, wu_ref, wd_ref, out_ref):
    e = pl.program_id(0)
    f = pl.program_id(1)

    @pl.when(jnp.logical_and(e == 0, f == 0))
    def _():
        out_ref[...] = jnp.zeros_like(out_ref)

    x = x_ref[...]
    g = jnp.dot(x, wg_ref[0], preferred_element_type=jnp.float32)
    s = g * (1.0 / (1.0 + jnp.exp(-g)))
    u = jnp.dot(x, wu_ref[0], preferred_element_type=jnp.float32)
    h = s * u
    p = jnp.dot(h, wd_ref[0], preferred_element_type=jnp.float32)
    comb = comb_ref[...]
    sel = jax.lax.broadcasted_iota(jnp.int32, comb.shape, 1) == e
    c = jnp.sum(jnp.where(sel, comb, 0.0), axis=1, keepdims=True)
    out_ref[...] += p * c


def _dense_ffn(x_flat, comb, Wg, Wu, Wd):
    return pl.pallas_call(
        _dense_ffn_body,
        grid=(NE, NFT),
        in_specs=[
            pl.BlockSpec((NTOK, EMBED), lambda e, f: (0, 0)),
            pl.BlockSpec((NTOK, NE), lambda e, f: (0, 0)),
            pl.BlockSpec((1, EMBED, FT), lambda e, f: (e, 0, f)),
            pl.BlockSpec((1, EMBED, FT), lambda e, f: (e, 0, f)),
            pl.BlockSpec((1, FT, EMBED), lambda e, f: (e, f, 0)),
        ],
        out_specs=pl.BlockSpec((NTOK, EMBED), lambda e, f: (0, 0)),
        out_shape=jax.ShapeDtypeStruct((NTOK, EMBED), jnp.float32),
    )(x_flat, comb, Wg, Wu, Wd)


def kernel(x, Wr, Wg, Wu, Wd):
    B, T, D = x.shape
    x_flat = x.reshape(B * T, D)
    comb, idx, w, loss = _router(x_flat, Wr)
    out_flat = _dense_ffn(x_flat, comb, Wg, Wu, Wd)
    return out_flat.reshape(B, T, D), loss.reshape(())
